# baseline (speedup 1.0000x reference)
import functools

import jax
import jax.numpy as jnp
from jax import lax
from jax.experimental import pallas as pl
from jax.experimental.pallas import tpu as pltpu
from jax.experimental.pallas import tpu_sc as plsc

_B = 4096
_L = 200
_D = 64
_NW = 32
_BT = _B // _NW

_mesh = plsc.VectorSubcoreMesh(core_axis_name="c", subcore_axis_name="s")


@functools.partial(
    pl.kernel,
    mesh=_mesh,
    out_type=jax.ShapeDtypeStruct((_B, _L, _D), jnp.float32),
    scratch_types=[
        pltpu.VMEM((_BT, 2, 128), jnp.int32),
        pltpu.VMEM((_L, _D), jnp.float32),
        pltpu.VMEM((_L, _D), jnp.float32),
        pltpu.SemaphoreType.DMA,
        pltpu.SemaphoreType.DMA,
        pltpu.SemaphoreType.DMA,
        pltpu.SemaphoreType.DMA,
    ],
    compiler_params=pltpu.CompilerParams(use_tc_tiling_on_sc=False),
)
def _gather_sc(xs_hbm, w_hbm, out_hbm, idx_v, g0, g1, sg0, sg1, sw0, sw1):
    wid = lax.axis_index("s") * 2 + lax.axis_index("c")
    b0 = wid * _BT

    pltpu.sync_copy(xs_hbm.at[pl.ds(b0, _BT)].at[:, pl.ds(0, 128)],
                    idx_v.at[:, 0])
    pltpu.sync_copy(xs_hbm.at[pl.ds(b0, _BT)].at[:, pl.ds(72, 128)],
                    idx_v.at[:, 1])

    def gathers(b, g, sem):
        pltpu.async_copy(w_hbm.at[idx_v.at[b].at[0]],
                         g.at[pl.ds(0, 128)], sem)
        pltpu.async_copy(w_hbm.at[idx_v.at[b].at[1]],
                         g.at[pl.ds(72, 128)], sem)

    def wait_gathers(g, sem):
        pltpu.make_async_copy(w_hbm.at[idx_v.at[0].at[0]],
                              g.at[pl.ds(0, 128)], sem).wait()
        pltpu.make_async_copy(w_hbm.at[idx_v.at[0].at[1]],
                              g.at[pl.ds(72, 128)], sem).wait()

    def wait_write(g, sem):
        pltpu.make_async_copy(g, out_hbm.at[b0], sem).wait()

    gathers(0, g0, sg0)
    gathers(1, g1, sg1)

    def peeled(b, g, sg, sw):
        wait_gathers(g, sg)
        pltpu.async_copy(g, out_hbm.at[b0 + b], sw)

    peeled(0, g0, sg0, sw0)
    peeled(1, g1, sg1, sw1)

    def body(k, carry):
        wait_write(g0, sw0)
        gathers(2 * k, g0, sg0)
        wait_gathers(g0, sg0)
        pltpu.async_copy(g0, out_hbm.at[b0 + 2 * k], sw0)
        wait_write(g1, sw1)
        gathers(2 * k + 1, g1, sg1)
        wait_gathers(g1, sg1)
        pltpu.async_copy(g1, out_hbm.at[b0 + 2 * k + 1], sw1)
        return carry

    lax.fori_loop(1, _BT // 2, body, 0)

    wait_write(g0, sw0)
    wait_write(g1, sw1)


def _tr_body(x_ref, o_ref):
    x = x_ref[...]
    xt = jnp.transpose(x, (1, 2, 0))
    o_ref[:, :, 0, :, :] = xt.reshape(8, 8, 8, 128)


_tr_call = pl.pallas_call(
    _tr_body,
    grid=(_L // 8, _B // 128),
    in_specs=[pl.BlockSpec((128, 8, _D), lambda l, bt: (bt, l, 0))],
    out_specs=pl.BlockSpec((8, 8, 1, 8, 128), lambda l, bt: (l, 0, bt, 0, 0)),
    out_shape=jax.ShapeDtypeStruct((_L, _D // 8, _B // 128, 8, 128), jnp.float32),
)


def kernel(xs, W):
    out_g = _gather_sc(xs.astype(jnp.int32), W)
    out5 = _tr_call(out_g)
    return out5.transpose(2, 4, 0, 1, 3).reshape(_B, _L, _D)


# speedup vs baseline: 1.0208x; 1.0208x over previous
import functools

import jax
import jax.numpy as jnp
from jax import lax
from jax.experimental import pallas as pl
from jax.experimental.pallas import tpu as pltpu
from jax.experimental.pallas import tpu_sc as plsc

_B = 4096
_L = 200
_D = 64
_NW = 32
_BT = _B // _NW

_mesh = plsc.VectorSubcoreMesh(core_axis_name="c", subcore_axis_name="s")


@functools.partial(
    pl.kernel,
    mesh=_mesh,
    out_type=jax.ShapeDtypeStruct((_B, _L, _D), jnp.float32),
    scratch_types=[
        pltpu.VMEM((_BT, 2, 128), jnp.int32),
        pltpu.VMEM((_L, _D), jnp.float32),
        pltpu.VMEM((_L, _D), jnp.float32),
        pltpu.SemaphoreType.DMA,
        pltpu.SemaphoreType.DMA,
        pltpu.SemaphoreType.DMA,
        pltpu.SemaphoreType.DMA,
    ],
    compiler_params=pltpu.CompilerParams(use_tc_tiling_on_sc=False),
)
def _gather_sc(xs_hbm, w_hbm, out_hbm, idx_v, g0, g1, sg0, sg1, sw0, sw1):
    wid = lax.axis_index("s") * 2 + lax.axis_index("c")
    b0 = wid * _BT

    pltpu.sync_copy(xs_hbm.at[pl.ds(b0, _BT)].at[:, pl.ds(0, 128)],
                    idx_v.at[:, 0])
    pltpu.sync_copy(xs_hbm.at[pl.ds(b0, _BT)].at[:, pl.ds(72, 128)],
                    idx_v.at[:, 1])

    def gathers(b, g, sem):
        pltpu.async_copy(w_hbm.at[idx_v.at[b].at[0]],
                         g.at[pl.ds(0, 128)], sem)
        pltpu.async_copy(w_hbm.at[idx_v.at[b].at[1]],
                         g.at[pl.ds(72, 128)], sem)

    def wait_gathers(g, sem):
        pltpu.make_async_copy(w_hbm.at[idx_v.at[0].at[0]],
                              g.at[pl.ds(0, 128)], sem).wait()
        pltpu.make_async_copy(w_hbm.at[idx_v.at[0].at[1]],
                              g.at[pl.ds(72, 128)], sem).wait()

    def wait_write(g, sem):
        pltpu.make_async_copy(g, out_hbm.at[b0], sem).wait()

    gathers(0, g0, sg0)
    gathers(1, g1, sg1)

    def half(b, g, sg, sw):
        wait_gathers(g, sg)
        pltpu.async_copy(g, out_hbm.at[b0 + b], sw)
        wait_write(g, sw)
        gathers(b + 2, g, sg)

    def body(k, carry):
        half(2 * k, g0, sg0, sw0)
        half(2 * k + 1, g1, sg1, sw1)
        return carry

    lax.fori_loop(0, _BT // 2 - 1, body, 0)

    wait_gathers(g0, sg0)
    pltpu.async_copy(g0, out_hbm.at[b0 + _BT - 2], sw0)
    wait_gathers(g1, sg1)
    pltpu.async_copy(g1, out_hbm.at[b0 + _BT - 1], sw1)
    wait_write(g0, sw0)
    wait_write(g1, sw1)


def _tr_body(x_ref, o_ref):
    x = x_ref[...]
    xt = jnp.transpose(x, (1, 2, 0))
    o_ref[:, :, 0, :, :] = xt.reshape(8, 8, 8, 128)


_tr_call = pl.pallas_call(
    _tr_body,
    grid=(_L // 8, _B // 128),
    in_specs=[pl.BlockSpec((128, 8, _D), lambda l, bt: (bt, l, 0))],
    out_specs=pl.BlockSpec((8, 8, 1, 8, 128), lambda l, bt: (l, 0, bt, 0, 0)),
    out_shape=jax.ShapeDtypeStruct((_L, _D // 8, _B // 128, 8, 128), jnp.float32),
)


def kernel(xs, W):
    out_g = _gather_sc(xs.astype(jnp.int32), W)
    out5 = _tr_call(out_g)
    return out5.transpose(2, 4, 0, 1, 3).reshape(_B, _L, _D)


# speedup vs baseline: 3.0838x; 3.0211x over previous
import functools

import jax
import jax.numpy as jnp
from jax import lax
from jax.experimental import pallas as pl
from jax.experimental.pallas import tpu as pltpu
from jax.experimental.pallas import tpu_sc as plsc

_B = 4096
_L = 200
_D = 64
_NW = 32
_BT = _B // _NW

_mesh = plsc.VectorSubcoreMesh(core_axis_name="c", subcore_axis_name="s")


@functools.partial(
    pl.kernel,
    mesh=_mesh,
    out_type=jax.ShapeDtypeStruct((_B, _L, _D), jnp.float32),
    scratch_types=[
        pltpu.VMEM((_BT, 2, 128), jnp.int32),
        pltpu.VMEM((_L, _D), jnp.float32),
        pltpu.VMEM((_L, _D), jnp.float32),
        pltpu.SemaphoreType.DMA,
        pltpu.SemaphoreType.DMA,
        pltpu.SemaphoreType.DMA,
        pltpu.SemaphoreType.DMA,
    ],
    compiler_params=pltpu.CompilerParams(use_tc_tiling_on_sc=False),
)
def _gather_sc(xs_hbm, w_hbm, out_hbm, idx_v, g0, g1, sg0, sg1, sw0, sw1):
    wid = lax.axis_index("s") * 2 + lax.axis_index("c")
    b0 = wid * _BT

    pltpu.sync_copy(xs_hbm.at[pl.ds(b0, _BT)].at[:, pl.ds(0, 128)],
                    idx_v.at[:, 0])
    pltpu.sync_copy(xs_hbm.at[pl.ds(b0, _BT)].at[:, pl.ds(72, 128)],
                    idx_v.at[:, 1])

    def gathers(b, g, sem):
        pltpu.async_copy(w_hbm.at[idx_v.at[b].at[0]],
                         g.at[pl.ds(0, 128)], sem)
        pltpu.async_copy(w_hbm.at[idx_v.at[b].at[1]],
                         g.at[pl.ds(72, 128)], sem)

    def wait_gathers(g, sem):
        pltpu.make_async_copy(w_hbm.at[idx_v.at[0].at[0]],
                              g.at[pl.ds(0, 128)], sem).wait()
        pltpu.make_async_copy(w_hbm.at[idx_v.at[0].at[1]],
                              g.at[pl.ds(72, 128)], sem).wait()

    def wait_write(g, sem):
        pltpu.make_async_copy(g, out_hbm.at[b0], sem).wait()

    gathers(0, g0, sg0)
    gathers(1, g1, sg1)

    def half(b, g, sg, sw):
        wait_gathers(g, sg)
        pltpu.async_copy(g, out_hbm.at[b0 + b], sw)
        wait_write(g, sw)
        gathers(b + 2, g, sg)

    def body(k, carry):
        half(2 * k, g0, sg0, sw0)
        half(2 * k + 1, g1, sg1, sw1)
        return carry

    lax.fori_loop(0, _BT // 2 - 1, body, 0)

    wait_gathers(g0, sg0)
    pltpu.async_copy(g0, out_hbm.at[b0 + _BT - 2], sw0)
    wait_gathers(g1, sg1)
    pltpu.async_copy(g1, out_hbm.at[b0 + _BT - 1], sw1)
    wait_write(g0, sw0)
    wait_write(g1, sw1)


def _tr_body(x_ref, o_ref):
    x = x_ref[...]
    x3 = x.reshape(128, 100, 128)
    xt = jnp.transpose(x3, (1, 2, 0))
    o_ref[:, :, 0, :, :] = xt.reshape(100, 2, 8, 8, 128).reshape(200, 8, 8, 128)


_tr_call = pl.pallas_call(
    _tr_body,
    grid=(_B // 128,),
    in_specs=[pl.BlockSpec((12800, 128), lambda bt: (bt, 0))],
    out_specs=pl.BlockSpec((_L, 8, 1, 8, 128), lambda bt: (0, 0, bt, 0, 0)),
    out_shape=jax.ShapeDtypeStruct((_L, _D // 8, _B // 128, 8, 128), jnp.float32),
    compiler_params=pltpu.CompilerParams(vmem_limit_bytes=100 * 1024 * 1024),
)


def kernel(xs, W):
    out_g = _gather_sc(xs.astype(jnp.int32), W)
    out5 = _tr_call(out_g.reshape(_B * _L // 2, 128))
    return out5.transpose(2, 4, 0, 1, 3).reshape(_B, _L, _D)
